# baseline (speedup 1.0000x reference)
import functools

import jax
import jax.numpy as jnp
from jax import lax
from jax.experimental import pallas as pl
from jax.experimental.pallas import tpu as pltpu
from jax.experimental.pallas import tpu_sc as plsc

_B, _L, _C = 4096, 20, 1000
_W = _C + 1
_N = _B * _L
_NC, _NS, _LANES = 2, 16, 16
_NW = _NC * _NS
_TPW = _N // _NW
_K = 32
_NCHUNK = _TPW // _K
_GROUPS = _K // _LANES


def _sc_body(act_hbm, dur_hbm, out_hbm, act_v, dur_v, buf0, buf1, sem0, sem1):
    wid = lax.axis_index("s") * _NC + lax.axis_index("c")
    base = wid * _TPW
    bufs = (buf0, buf1)
    sems = (sem0, sem1)

    pltpu.sync_copy(act_hbm.at[pl.ds(base, _TPW)], act_v)
    pltpu.sync_copy(dur_hbm.at[pl.ds(base, _TPW)], dur_v)

    zeros16 = jnp.zeros((_LANES,), jnp.float32)
    ones16 = jnp.ones((_LANES,), jnp.float32)
    lane = lax.iota(jnp.int32, _LANES)

    def zero_fill(i, carry):
        o = i * _LANES
        buf0[pl.ds(o, _LANES)] = zeros16
        buf1[pl.ds(o, _LANES)] = zeros16
        return carry

    lax.fori_loop(0, _K * _W // _LANES, zero_fill, 0)

    def outer(go, carry):
        for b in range(2):
            chunk = go * 2 + b
            buf, sem = bufs[b], sems[b]

            @pl.when(go >= 1)
            def _reuse():
                prev = chunk - 2
                pltpu.make_async_copy(
                    buf,
                    out_hbm.at[pl.ds((base + prev * _K) * _W, _K * _W)],
                    sem,
                ).wait()
                for j in range(_GROUPS):
                    row = j * _LANES + lane
                    old_act = act_v[pl.ds(prev * _K + j * _LANES, _LANES)]
                    plsc.store_scatter(buf, [row * _W + old_act], zeros16)

            for j in range(_GROUPS):
                row = j * _LANES + lane
                new_act = act_v[pl.ds(chunk * _K + j * _LANES, _LANES)]
                new_dur = dur_v[pl.ds(chunk * _K + j * _LANES, _LANES)]
                plsc.store_scatter(buf, [row * _W + new_act], ones16)
                plsc.store_scatter(buf, [row * _W + _C], new_dur)

            pltpu.make_async_copy(
                buf,
                out_hbm.at[pl.ds((base + chunk * _K) * _W, _K * _W)],
                sem,
            ).start()
        return carry

    lax.fori_loop(0, _NCHUNK // 2, outer, 0)

    for b in range(2):
        chunk = _NCHUNK - 2 + b
        pltpu.make_async_copy(
            bufs[b],
            out_hbm.at[pl.ds((base + chunk * _K) * _W, _K * _W)],
            sems[b],
        ).wait()


def kernel(x):
    act = x[..., 0].astype(jnp.int32).reshape(_N)
    dur = x[..., 1].reshape(_N)
    mesh = plsc.VectorSubcoreMesh(core_axis_name="c", subcore_axis_name="s")
    run = functools.partial(
        pl.kernel,
        mesh=mesh,
        out_type=jax.ShapeDtypeStruct((_N * _W,), jnp.float32),
        scratch_types=[
            pltpu.VMEM((_TPW,), jnp.int32),
            pltpu.VMEM((_TPW,), jnp.float32),
            pltpu.VMEM((_K * _W,), jnp.float32),
            pltpu.VMEM((_K * _W,), jnp.float32),
            pltpu.SemaphoreType.DMA,
            pltpu.SemaphoreType.DMA,
        ],
        compiler_params=pltpu.CompilerParams(needs_layout_passes=False),
    )(_sc_body)
    out = run(act, dur)
    return out.reshape(_B, _L, _W)
